# baseline (speedup 1.0000x reference)
import functools
import jax
import jax.numpy as jnp
from jax import lax
from jax.experimental import pallas as pl
from jax.experimental.pallas import tpu as pltpu
from jax.experimental.pallas import tpu_sc as plsc

K = 1024
D = 64
BETA = 0.25
BLK = 512

NC, NS, L = 2, 16, 16
NW = NC * NS
N = 16 * 32 * 32
PPW = N // NW
CH = PPW // 128


def _dist_argmin_block(flat_ref, emb_ref, idx_ref, loss_ref, se_ref):
    i = pl.program_id(0)
    x = flat_ref[...]
    e = emb_ref[...]

    @pl.when(i == 0)
    def _init():
        se_ref[...] = jnp.sum(e ** 2, axis=1)[None, :]
        loss_ref[...] = jnp.zeros_like(loss_ref)

    m = lax.dot_general(x, e, (((1,), (1,)), ((), ())),
                        preferred_element_type=jnp.float32)
    sx = jnp.sum(x ** 2, axis=1, keepdims=True)
    dist = sx + se_ref[...] - 2.0 * m
    minv = jnp.min(dist, axis=1, keepdims=True)
    ids = lax.broadcasted_iota(jnp.int32, (BLK, K), 1)
    idx_ref[...] = jnp.min(jnp.where(dist == minv, ids, K), axis=1)
    loss_ref[...] += jnp.reshape(jnp.sum(minv), (1, 1))


def _sc_gather_body(emb_hbm, idx_hbm, out_hbm, idx_v, rows_v, sem):
    wid = lax.axis_index("s") * NC + lax.axis_index("c")
    pltpu.sync_copy(idx_hbm.at[wid], idx_v)
    copies = [
        pltpu.async_copy(emb_hbm.at[idx_v.at[j]],
                         rows_v.at[pl.ds(j * 128, 128)], sem)
        for j in range(CH)
    ]
    for c in copies:
        c.wait()
    pltpu.sync_copy(rows_v, out_hbm.at[wid])


_sc_gather = pl.kernel(
    _sc_gather_body,
    out_type=jax.ShapeDtypeStruct((NW, PPW, D), jnp.float32),
    mesh=plsc.VectorSubcoreMesh(core_axis_name="c", subcore_axis_name="s",
                                num_cores=NC, num_subcores=NS),
    scratch_types=[
        pltpu.VMEM((CH, 128), jnp.int32),
        pltpu.VMEM((PPW, D), jnp.float32),
        pltpu.SemaphoreType.DMA,
    ],
    compiler_params=pltpu.CompilerParams(use_tc_tiling_on_sc=False),
)


def kernel(latents, embedding_weight):
    lat = jnp.transpose(latents, (0, 2, 3, 1))
    shp = lat.shape
    flat = lat.reshape(-1, D)
    n = flat.shape[0]
    idx, loss = pl.pallas_call(
        _dist_argmin_block,
        grid=(n // BLK,),
        in_specs=[pl.BlockSpec((BLK, D), lambda i: (i, 0)),
                  pl.BlockSpec((K, D), lambda i: (0, 0))],
        out_specs=[pl.BlockSpec((BLK,), lambda i: (i,)),
                   pl.BlockSpec((1, 1), lambda i: (0, 0))],
        out_shape=[jax.ShapeDtypeStruct((n,), jnp.int32),
                   jax.ShapeDtypeStruct((1, 1), jnp.float32)],
        scratch_shapes=[pltpu.VMEM((1, K), jnp.float32)],
    )(flat, embedding_weight)
    q = _sc_gather(embedding_weight, idx.reshape(NW, CH, 128))
    l = loss[0, 0] / (n * D)
    out = jnp.transpose(q.reshape(shp), (0, 3, 1, 2))
    return (out, l * BETA, l)


# speedup vs baseline: 1.3821x; 1.3821x over previous
import jax
import jax.numpy as jnp
from jax import lax
from jax.experimental import pallas as pl
from jax.experimental.pallas import tpu as pltpu

K = 1024
D = 64
BETA = 0.25
P = 512


def _vq_block(x_ref, emb_ref, out_ref, loss_ref, se_ref):
    first = (pl.program_id(0) == 0) & (pl.program_id(1) == 0)
    x = x_ref[0]
    e = emb_ref[...]

    @pl.when(first)
    def _init():
        se_ref[...] = jnp.sum(e ** 2, axis=1, keepdims=True)
        loss_ref[...] = jnp.zeros_like(loss_ref)

    m = lax.dot_general(e, x, (((1,), (0,)), ((), ())),
                        preferred_element_type=jnp.float32)
    sx = jnp.sum(x ** 2, axis=0, keepdims=True)
    dist = sx + se_ref[...] - 2.0 * m
    minv = jnp.min(dist, axis=0, keepdims=True)
    ids = lax.broadcasted_iota(jnp.int32, (K, P), 0)
    idx = jnp.min(jnp.where(dist == minv, ids, K), axis=0)
    oh = (ids == idx[None, :]).astype(jnp.float32)
    out_ref[0] = lax.dot_general(e, oh, (((0,), (0,)), ((), ())),
                                 preferred_element_type=jnp.float32)
    loss_ref[...] += jnp.reshape(jnp.sum(minv), (1, 1))


def kernel(latents, embedding_weight):
    b, c, h, w = latents.shape
    n = b * h * w
    cols = latents.reshape(b, c, h * w)
    out_cols, loss = pl.pallas_call(
        _vq_block,
        grid=(b, h * w // P),
        in_specs=[pl.BlockSpec((1, D, P), lambda i, j: (i, 0, j)),
                  pl.BlockSpec((K, D), lambda i, j: (0, 0))],
        out_specs=[pl.BlockSpec((1, D, P), lambda i, j: (i, 0, j)),
                   pl.BlockSpec((1, 1), lambda i, j: (0, 0))],
        out_shape=[jax.ShapeDtypeStruct((b, D, h * w), jnp.float32),
                   jax.ShapeDtypeStruct((1, 1), jnp.float32)],
        scratch_shapes=[pltpu.VMEM((K, 1), jnp.float32)],
    )(cols, embedding_weight)
    l = loss[0, 0] / (n * D)
    return (out_cols.reshape(b, c, h, w), l * BETA, l)


# speedup vs baseline: 1.6192x; 1.1715x over previous
import jax
import jax.numpy as jnp
from jax import lax
from jax.experimental import pallas as pl
from jax.experimental.pallas import tpu as pltpu

K = 1024
D = 64
BETA = 0.25
P = 1024


def _vq_block(x_ref, emb_ref, out_ref, loss_ref, se_ref):
    first = (pl.program_id(0) == 0) & (pl.program_id(1) == 0)
    x = x_ref[0]
    e = emb_ref[...]

    @pl.when(first)
    def _init():
        se_ref[...] = jnp.sum(e ** 2, axis=1, keepdims=True)
        loss_ref[...] = jnp.zeros_like(loss_ref)

    m2 = lax.dot_general(e + e, x, (((1,), (0,)), ((), ())),
                         preferred_element_type=jnp.float32)
    sx = jnp.sum(x ** 2, axis=0, keepdims=True)
    dist = sx + se_ref[...] - m2
    minv = jnp.min(dist, axis=0, keepdims=True)
    ids = lax.broadcasted_iota(jnp.int32, (K, P), 0)
    idx = jnp.min(jnp.where(dist == minv, ids, K), axis=0)
    oh = (ids == idx[None, :]).astype(jnp.float32)
    out_ref[0] = lax.dot_general(e, oh, (((0,), (0,)), ((), ())),
                                 preferred_element_type=jnp.float32)
    loss_ref[...] += jnp.reshape(jnp.sum(minv), (1, 1))


def kernel(latents, embedding_weight):
    b, c, h, w = latents.shape
    n = b * h * w
    cols = latents.reshape(b, c, h * w)
    out_cols, loss = pl.pallas_call(
        _vq_block,
        grid=(b, h * w // P),
        in_specs=[pl.BlockSpec((1, D, P), lambda i, j: (i, 0, j)),
                  pl.BlockSpec((K, D), lambda i, j: (0, 0))],
        out_specs=[pl.BlockSpec((1, D, P), lambda i, j: (i, 0, j)),
                   pl.BlockSpec((1, 1), lambda i, j: (0, 0))],
        out_shape=[jax.ShapeDtypeStruct((b, D, h * w), jnp.float32),
                   jax.ShapeDtypeStruct((1, 1), jnp.float32)],
        scratch_shapes=[pltpu.VMEM((K, 1), jnp.float32)],
    )(cols, embedding_weight)
    l = loss[0, 0] / (n * D)
    return (out_cols.reshape(b, c, h, w), l * BETA, l)


# speedup vs baseline: 1.6955x; 1.0471x over previous
import jax
import jax.numpy as jnp
from jax import lax
from jax.experimental import pallas as pl
from jax.experimental.pallas import tpu as pltpu

K = 1024
D = 64
BETA = 0.25
P = 1024
IB = 2


def _vq_block(x_ref, emb_ref, out_ref, loss_ref, se_ref):
    first = (pl.program_id(0) == 0) & (pl.program_id(1) == 0)
    e = emb_ref[...]

    @pl.when(first)
    def _init():
        se_ref[...] = jnp.sum(e ** 2, axis=1, keepdims=True)
        loss_ref[...] = jnp.zeros_like(loss_ref)

    e2 = e + e
    for sub in range(IB):
        x = x_ref[sub]
        m2 = lax.dot_general(e2, x, (((1,), (0,)), ((), ())),
                             preferred_element_type=jnp.float32)
        sx = jnp.sum(x ** 2, axis=0, keepdims=True)
        dist = sx + se_ref[...] - m2
        minv = jnp.min(dist, axis=0, keepdims=True)
        ids = lax.broadcasted_iota(jnp.int32, (K, P), 0)
        idx = jnp.min(jnp.where(dist == minv, ids, K), axis=0)
        oh = (ids == idx[None, :]).astype(jnp.float32)
        out_ref[sub] = lax.dot_general(e, oh, (((0,), (0,)), ((), ())),
                                       preferred_element_type=jnp.float32)
        loss_ref[...] += jnp.reshape(jnp.sum(minv), (1, 1))


def kernel(latents, embedding_weight):
    b, c, h, w = latents.shape
    n = b * h * w
    cols = latents.reshape(b, c, h * w)
    out_cols, loss = pl.pallas_call(
        _vq_block,
        grid=(b // IB, h * w // P),
        in_specs=[pl.BlockSpec((IB, D, P), lambda i, j: (i, 0, j)),
                  pl.BlockSpec((K, D), lambda i, j: (0, 0))],
        out_specs=[pl.BlockSpec((IB, D, P), lambda i, j: (i, 0, j)),
                   pl.BlockSpec((1, 1), lambda i, j: (0, 0))],
        out_shape=[jax.ShapeDtypeStruct((b, D, h * w), jnp.float32),
                   jax.ShapeDtypeStruct((1, 1), jnp.float32)],
        scratch_shapes=[pltpu.VMEM((K, 1), jnp.float32)],
    )(cols, embedding_weight)
    l = loss[0, 0] / (n * D)
    return (out_cols.reshape(b, c, h, w), l * BETA, l)
